# Initial kernel scaffold; baseline (speedup 1.0000x reference)
import jax
import jax.numpy as jnp
from jax.experimental import pallas as pl


def kernel(token_ids, weight):
    raise NotImplementedError("write your pallas kernel here")


# speedup vs baseline: 1.1054x; 1.1054x over previous
import functools

import jax
import jax.numpy as jnp
from jax import lax
from jax.experimental import pallas as pl
from jax.experimental.pallas import tpu as pltpu
from jax.experimental.pallas import tpu_sc as plsc

_NUM_EMB = 1000000
_D = 32
_TOTAL = 16384 * 50
_NW = 32
_G = 128
_ROWS = _TOTAL // _G
_ROWS_W = _ROWS // _NW
_K = 10
_CHUNKS = _ROWS_W // _K
_CH = _K * _G

_mesh = plsc.VectorSubcoreMesh(core_axis_name="c", subcore_axis_name="s")


@functools.partial(
    pl.kernel,
    mesh=_mesh,
    compiler_params=pltpu.CompilerParams(use_tc_tiling_on_sc=False),
    out_type=jax.ShapeDtypeStruct((_TOTAL, _D), jnp.float32),
    scratch_types=[
        pltpu.VMEM((_ROWS_W, _G), jnp.int32),
        pltpu.VMEM((_CH, _D), jnp.float32),
        pltpu.SemaphoreType.DMA,
    ],
)
def _emb_lookup(table_hbm, idx_hbm, out_hbm, idx_v, rows_v, sem):
    wid = lax.axis_index("s") * 2 + lax.axis_index("c")
    row_base = wid * _ROWS_W
    pltpu.sync_copy(idx_hbm.at[pl.ds(row_base, _ROWS_W)], idx_v)
    out_base = wid * (_ROWS_W * _G)

    def chunk(s, carry):
        copies = []
        for j in range(_K):
            cp = pltpu.async_copy(
                table_hbm.at[idx_v.at[s * _K + j]],
                rows_v.at[pl.ds(j * _G, _G)],
                sem,
            )
            copies.append(cp)
        for cp in copies:
            cp.wait()
        pltpu.sync_copy(rows_v, out_hbm.at[pl.ds(out_base + s * _CH, _CH)])
        return carry

    lax.fori_loop(0, _CHUNKS, chunk, 0)


def kernel(token_ids, weight):
    idx = token_ids.reshape(_ROWS, _G).astype(jnp.int32)
    out = _emb_lookup(weight, idx)
    return out.reshape(token_ids.shape + (_D,))
